# baseline (speedup 1.0000x reference)
import functools

import jax
import jax.numpy as jnp
from jax import lax
from jax.experimental import pallas as pl
from jax.experimental.pallas import tpu as pltpu
from jax.experimental.pallas import tpu_sc as plsc

NODES_PER = 200
N_GRAPHS_C = 50
EDGES_PER_G = 3200
NP_PAD = 208
HD = 128


def _bfs_order_sc(edge_index):
    n = NODES_PER
    e = EDGES_PER_G
    ng = N_GRAPHS_C
    mesh = plsc.VectorSubcoreMesh(core_axis_name="c", subcore_axis_name="s")
    nw = 32
    rounds = (ng + nw - 1) // nw

    @functools.partial(
        pl.kernel,
        out_type=jax.ShapeDtypeStruct((ng * n,), jnp.int32),
        mesh=mesh,
        compiler_params=pltpu.CompilerParams(needs_layout_passes=False),
        scratch_types=[
            pltpu.VMEM((e,), jnp.int32),
            pltpu.VMEM((e,), jnp.int32),
            pltpu.VMEM((e + 16,), jnp.int32),
            pltpu.VMEM((NP_PAD + 32,), jnp.int32),
            pltpu.SMEM((n + 1,), jnp.int32),
            pltpu.SMEM((n,), jnp.int32),
            pltpu.SMEM((n,), jnp.int32),
        ],
    )
    def k(eidx, out, src_v, dst_v, csr_v, ord_v, ptr_s, deg_s, vis_s):
        cid = lax.axis_index("c")
        sid = lax.axis_index("s")
        wid = sid * 2 + cid

        lane0 = jnp.arange(16, dtype=jnp.int32) == 0

        def scat1(ref, pos, val):
            pvec = jnp.broadcast_to(pos, (16,)).astype(jnp.int32)
            vvec = jnp.broadcast_to(val, (16,)).astype(jnp.int32)
            plsc.store_scatter(ref, [pvec], vvec, mask=lane0)

        def do_graph(g):
            base = g * n
            pltpu.sync_copy(eidx.at[0, pl.ds(g * e, e)], src_v)
            pltpu.sync_copy(eidx.at[1, pl.ds(g * e, e)], dst_v)

            def zblk(v, _):
                deg_s[v] = jnp.int32(0)
                vis_s[v] = jnp.int32(0)
                return 0
            lax.fori_loop(0, n, zblk, 0)

            def hist(i, _):
                s16 = src_v[pl.ds(i * 16, 16)] - base
                for l in range(16):
                    b = s16[l]
                    deg_s[b] = deg_s[b] + 1
                return 0
            lax.fori_loop(0, e // 16, hist, 0)

            def amax(v, carry):
                best, s0 = carry
                d = deg_s[v]
                better = d > best
                return (jnp.where(better, d, best),
                        jnp.where(better, v, s0))
            _, s0 = lax.fori_loop(0, n, amax,
                                  (jnp.int32(-1), jnp.int32(0)))

            ptr_s[0] = jnp.int32(0)

            def pfx(v, c):
                d = deg_s[v]
                c2 = c + d
                ptr_s[v + 1] = c2
                deg_s[v] = c2 - d
                return c2
            lax.fori_loop(0, n, pfx, jnp.int32(0))

            def place(i, _):
                s16 = src_v[pl.ds(i * 16, 16)] - base
                d16 = dst_v[pl.ds(i * 16, 16)] - base
                for l in range(16):
                    b = s16[l]
                    pos = deg_s[b]
                    scat1(csr_v, pos, d16[l])
                    deg_s[b] = pos + 1
                return 0
            lax.fori_loop(0, e // 16, place, 0)

            vis_s[s0] = jnp.int32(1)
            scat1(ord_v, 0, s0)

            def visit(i, tail):
                u_raw = ord_v[pl.ds(i, 16)][0]
                active = i < tail
                u = jnp.where(active, u_raw, 0)
                lo = jnp.where(active, ptr_s[u], 0)
                hi = jnp.where(active, ptr_s[u + 1], 0)

                def scan_edge(carry):
                    ei, t = carry
                    v = csr_v[pl.ds(ei, 16)][0]
                    was = vis_s[v]
                    vis_s[v] = jnp.int32(1)
                    slot = jnp.where(was == 1, jnp.int32(n), t)
                    scat1(ord_v, slot, v)
                    return (ei + 1, t + (1 - was))

                _, tail2 = lax.while_loop(lambda c: c[0] < hi, scan_edge,
                                          (lo, tail))
                return tail2

            tail = lax.fori_loop(0, n, visit, jnp.int32(1))

            def fill(v, t):
                was = vis_s[v]
                slot = jnp.where(was == 1, jnp.int32(n), t)
                scat1(ord_v, slot, v)
                return t + (1 - was)
            lax.fori_loop(0, n, fill, tail)

            pltpu.sync_copy(ord_v.at[pl.ds(0, n)], out.at[pl.ds(g * n, n)])

        for r in range(rounds):
            g = wid + r * nw

            @pl.when(g < ng)
            def _():
                do_graph(g)

    return k(edge_index)


def _gine_msg_sc(h, e_feat, src_arr, dst_arr):
    n = NODES_PER
    epg = EDGES_PER_G
    ng = N_GRAPHS_C
    mesh = plsc.VectorSubcoreMesh(core_axis_name="c", subcore_axis_name="s")
    nw = 32
    rounds = (ng + nw - 1) // nw
    ce = 80
    nch = epg // ce

    @functools.partial(
        pl.kernel,
        out_type=jax.ShapeDtypeStruct((ng * n, HD), jnp.float32),
        mesh=mesh,
        compiler_params=pltpu.CompilerParams(needs_layout_passes=False),
        scratch_types=[
            pltpu.VMEM((ce, HD), jnp.float32),
            pltpu.VMEM((ce, HD), jnp.float32),
            pltpu.VMEM((ce, HD), jnp.float32),
            pltpu.VMEM((ce, HD), jnp.float32),
            pltpu.VMEM((ce, HD), jnp.float32),
            pltpu.VMEM((ce, HD), jnp.float32),
            pltpu.VMEM((n, HD), jnp.float32),
            pltpu.VMEM((ce,), jnp.int32),
            pltpu.VMEM((ce,), jnp.int32),
            pltpu.VMEM((ce,), jnp.int32),
            pltpu.VMEM((ce,), jnp.int32),
            pltpu.VMEM((ce,), jnp.int32),
            pltpu.VMEM((ce,), jnp.int32),
            pltpu.VMEM_SHARED((16 * n, HD), jnp.float32),
            pltpu.SemaphoreType.DMA,
            pltpu.SemaphoreType.DMA,
            pltpu.SemaphoreType.DMA,
            pltpu.SemaphoreType.DMA,
            pltpu.SemaphoreType.DMA,
            pltpu.SemaphoreType.DMA,
        ],
    )
    def k(h_hbm, e_hbm, src_h, dst_h, out, hg0, hg1, e0, e1, msg0, msg1, zero_v,
          sidx0, sidx1, draw0, draw1, didx0, didx1, ag_sh,
          semi0, semi1, semg0, semg1, sems0, sems1):
        cid = lax.axis_index("c")
        sid = lax.axis_index("s")
        wid = sid * 2 + cid
        hg_v = (hg0, hg1)
        e_v = (e0, e1)
        msg_v = (msg0, msg1)
        sidx_v = (sidx0, sidx1)
        draw_v = (draw0, draw1)
        didx_v = (didx0, didx1)
        semi = (semi0, semi1)
        semg = (semg0, semg1)
        sems = (sems0, sems1)

        zeros16 = jnp.zeros((16,), jnp.float32)

        @plsc.parallel_loop(0, n)
        def zrow(r):
            for c in range(HD // 16):
                zero_v[r, pl.ds(c * 16, 16)] = zeros16

        def issue_in(g, cidx, b):
            eoff = g * epg + cidx * ce
            pltpu.async_copy(src_h.at[pl.ds(eoff, ce)], sidx_v[b],
                             semi[b])
            pltpu.async_copy(dst_h.at[pl.ds(eoff, ce)], draw_v[b],
                             semi[b])
            pltpu.async_copy(e_hbm.at[pl.ds(eoff, ce)], e_v[b], semi[b])

        def wait_in(g, cidx, b):
            eoff = g * epg + cidx * ce
            pltpu.make_async_copy(src_h.at[pl.ds(eoff, ce)],
                                  sidx_v[b], semi[b]).wait()
            pltpu.make_async_copy(dst_h.at[pl.ds(eoff, ce)],
                                  draw_v[b], semi[b]).wait()
            pltpu.make_async_copy(e_hbm.at[pl.ds(eoff, ce)], e_v[b],
                                  semi[b]).wait()

        def wait_scat(b):
            pltpu.make_async_copy(msg_v[b], ag_sh.at[pl.ds(0, ce)],
                                  sems[b]).wait()

        def issue_gather(b):
            pltpu.async_copy(h_hbm.at[sidx_v[b]], hg_v[b], semg[b])

        def wait_gather(b):
            pltpu.make_async_copy(h_hbm.at[sidx_v[b]], hg_v[b],
                                  semg[b]).wait()

        def do_graph(g):
            base = g * n
            arow = sid * n
            pltpu.sync_copy(zero_v, ag_sh.at[pl.ds(arow, n)])
            shift = arow - base

            issue_in(g, 0, 0)
            wait_in(g, 0, 0)
            issue_gather(0)
            issue_in(g, 1, 1)

            def chunk(cidx, _):
                for b in range(2):

                    @pl.when((cidx & 1) == b)
                    def _():
                        @pl.when(cidx + 1 < nch)
                        def _():
                            wait_in(g, cidx + 1, 1 - b)
                            issue_gather(1 - b)

                        @plsc.parallel_loop(0, ce // 16)
                        def dloc(i):
                            sl = pl.ds(i * 16, 16)
                            didx_v[b][sl] = draw_v[b][sl] + shift

                        @pl.when(cidx >= 2)
                        def _():
                            wait_scat(b)

                        wait_gather(b)

                        @plsc.parallel_loop(0, ce)
                        def mrow(r):
                            for c in range(HD // 16):
                                sl = pl.ds(c * 16, 16)
                                msg_v[b][r, sl] = jnp.maximum(
                                    hg_v[b][r, sl] + e_v[b][r, sl], 0.0)

                        pltpu.async_copy(msg_v[b],
                                         ag_sh.at[didx_v[b]],
                                         sems[b], add=True)

                        @pl.when(cidx + 2 < nch)
                        def _():
                            issue_in(g, cidx + 2, b)
                return 0
            lax.fori_loop(0, nch, chunk, 0)

            wait_scat(nch & 1)
            wait_scat(1 - (nch & 1))
            pltpu.sync_copy(ag_sh.at[pl.ds(arow, n)], out.at[pl.ds(base, n)])

        for r in range(rounds):
            g = wid + r * nw

            @pl.when(g < ng)
            def _():
                do_graph(g)

    return k(h, e_feat, src_arr, dst_arr)


def _gather_rows_sc(table, idx):
    nrow, d = idx.shape[0], table.shape[1]
    mesh = plsc.VectorSubcoreMesh(core_axis_name="c", subcore_axis_name="s")
    nw = 32
    rows_per = nrow // nw
    chunk = 80
    nch = rows_per // chunk

    @functools.partial(
        pl.kernel,
        out_type=jax.ShapeDtypeStruct((nrow, d), jnp.float32),
        mesh=mesh,
        compiler_params=pltpu.CompilerParams(needs_layout_passes=False),
        scratch_types=[
            pltpu.VMEM((chunk,), jnp.int32),
            pltpu.VMEM((chunk, d), jnp.float32),
            pltpu.SemaphoreType.DMA,
        ],
    )
    def k(tab, iv, out, idx_v, rows_v, sem):
        cid = lax.axis_index("c")
        sid = lax.axis_index("s")
        wid = sid * 2 + cid
        base = wid * rows_per

        def body(c, _):
            off = base + c * chunk
            pltpu.sync_copy(iv.at[pl.ds(off, chunk)], idx_v)
            pltpu.async_copy(tab.at[idx_v], rows_v, sem).wait()
            pltpu.sync_copy(rows_v, out.at[pl.ds(off, chunk)])
            return 0
        lax.fori_loop(0, nch, body, 0)

    return k(table, idx)


def _edge_mlp_tc(ea, We1, be1, We2, be2):
    E = ea.shape[0]
    blk = 2000
    grid = E // blk

    def body(ea_ref, w1_ref, b1_ref, w2_ref, b2_ref, out_ref):
        a = jnp.maximum(
            jnp.dot(ea_ref[...], w1_ref[...],
                    preferred_element_type=jnp.float32) + b1_ref[...], 0.0)
        out_ref[...] = (
            jnp.dot(a, w2_ref[...], preferred_element_type=jnp.float32)
            + b2_ref[...])

    return pl.pallas_call(
        body,
        grid=(grid,),
        in_specs=[
            pl.BlockSpec((blk, 16), lambda i: (i, 0)),
            pl.BlockSpec((16, HD), lambda i: (0, 0)),
            pl.BlockSpec((1, HD), lambda i: (0, 0)),
            pl.BlockSpec((HD, HD), lambda i: (0, 0)),
            pl.BlockSpec((1, HD), lambda i: (0, 0)),
        ],
        out_specs=pl.BlockSpec((blk, HD), lambda i: (i, 0)),
        out_shape=jax.ShapeDtypeStruct((E, HD), jnp.float32),
    )(ea, We1, be1[None, :], We2, be2[None, :])


def _node_in_tc(x, W, b):
    N = x.shape[0]

    def body(x_ref, w_ref, b_ref, out_ref):
        out_ref[...] = jnp.dot(
            x_ref[...], w_ref[...],
            preferred_element_type=jnp.float32) + b_ref[...]

    return pl.pallas_call(
        body,
        grid=(5,),
        in_specs=[
            pl.BlockSpec((N // 5, HD), lambda i: (i, 0)),
            pl.BlockSpec((HD, HD), lambda i: (0, 0)),
            pl.BlockSpec((1, HD), lambda i: (0, 0)),
        ],
        out_specs=pl.BlockSpec((N // 5, HD), lambda i: (i, 0)),
        out_shape=jax.ShapeDtypeStruct((N, HD), jnp.float32),
    )(x, W, b[None, :])


def _gine_layer_tc(h, aggr, c_eps, W1, b1, W2, b2, gamma, beta):
    N = h.shape[0]

    def body(h_ref, a_ref, ce_ref, w1_ref, b1_ref, w2_ref, b2_ref,
             g_ref, be_ref, out_ref):
        ce = ce_ref[0]
        out = ce * h_ref[...] + a_ref[...]
        a1 = jnp.maximum(
            jnp.dot(out, w1_ref[...],
                    preferred_element_type=jnp.float32) + b1_ref[...], 0.0)
        h2 = jnp.dot(a1, w2_ref[...],
                     preferred_element_type=jnp.float32) + b2_ref[...]
        mean = jnp.mean(h2, axis=0, keepdims=True)
        var = jnp.mean(jnp.square(h2), axis=0, keepdims=True) - mean * mean
        h2n = (g_ref[...] * (h2 - mean) * jax.lax.rsqrt(var + 1e-5)
               + be_ref[...])
        out_ref[...] = jnp.maximum(h2n, 0.0) + h_ref[...]

    return pl.pallas_call(
        body,
        in_specs=[
            pl.BlockSpec(memory_space=pltpu.VMEM),
            pl.BlockSpec(memory_space=pltpu.VMEM),
            pl.BlockSpec(memory_space=pltpu.SMEM),
            pl.BlockSpec(memory_space=pltpu.VMEM),
            pl.BlockSpec(memory_space=pltpu.VMEM),
            pl.BlockSpec(memory_space=pltpu.VMEM),
            pl.BlockSpec(memory_space=pltpu.VMEM),
            pl.BlockSpec(memory_space=pltpu.VMEM),
            pl.BlockSpec(memory_space=pltpu.VMEM),
        ],
        out_shape=jax.ShapeDtypeStruct((N, HD), jnp.float32),
    )(h, aggr, c_eps, W1, b1[None, :], W2, b2[None, :],
      gamma[None, :], beta[None, :])


def _gi_tc(h_ord_t, pe, W_ihT, b_ih):
    TB = h_ord_t.shape[0]
    T = pe.shape[0]
    B = TB // T

    def body(x_ref, pe_ref, w_ref, b_ref, y1_ref, ype_ref, gm_ref):
        x = x_ref[...]
        y1_ref[...] = jnp.dot(
            x, w_ref[...], preferred_element_type=jnp.float32) + b_ref[...]
        ype_ref[...] = jnp.dot(
            pe_ref[...], w_ref[...], preferred_element_type=jnp.float32)
        gm_ref[...] = jnp.mean(x.reshape(T, B, HD), axis=0)

    return pl.pallas_call(
        body,
        out_shape=(
            jax.ShapeDtypeStruct((TB, 3 * HD), jnp.float32),
            jax.ShapeDtypeStruct((T, 3 * HD), jnp.float32),
            jax.ShapeDtypeStruct((B, HD), jnp.float32),
        ),
    )(h_ord_t, pe, W_ihT, b_ih[None, :])


def _gru_tc(Y1, Ype, gmean, W_hhT, b_hh, W_out, b_out):
    T, B = Y1.shape[0], Y1.shape[1]

    def body(y1_ref, ype_ref, gm_ref, whh_ref, bhh_ref, wo_ref, bo_ref,
             out_ref):
        whh = whh_ref[...]
        bhh = bhh_ref[...]

        def step(t, carry):
            h, s = carry
            gi = (y1_ref[pl.ds(t, 1)].reshape(B, 3 * HD)
                  + ype_ref[pl.ds(t, 1)])
            gh = jnp.dot(h, whh, preferred_element_type=jnp.float32) + bhh
            ir, iz, inn = gi[:, :HD], gi[:, HD:2 * HD], gi[:, 2 * HD:]
            hr, hz, hn = gh[:, :HD], gh[:, HD:2 * HD], gh[:, 2 * HD:]
            r = jax.nn.sigmoid(ir + hr)
            z = jax.nn.sigmoid(iz + hz)
            nn_ = jnp.tanh(inn + r * hn)
            hnew = (1.0 - z) * nn_ + z * h
            return (hnew, s + hnew)

        h0 = jnp.zeros((B, HD), jnp.float32)
        h_last, s = lax.fori_loop(0, T, step, (h0, h0))
        pooled = jnp.concatenate(
            [h_last, s * (1.0 / T), gm_ref[...]], axis=1)
        out_ref[...] = jnp.dot(
            pooled, wo_ref[...], preferred_element_type=jnp.float32
        ) + bo_ref[...]

    return pl.pallas_call(
        body,
        out_shape=jax.ShapeDtypeStruct((B, HD), jnp.float32),
    )(Y1, Ype, gmean, W_hhT, b_hh[None, :], W_out, b_out[None, :])


def kernel(x, edge_index, edge_attr, batch_vec, params):
    p = params
    del batch_vec
    n = x.shape[0]
    B = N_GRAPHS_C
    T = NODES_PER
    eidx = edge_index.astype(jnp.int32)
    src = eidx[0]
    dst = eidx[1]

    e = _edge_mlp_tc(edge_attr, p['We1'], p['be1'], p['We2'], p['be2'])
    h = _node_in_tc(x, p['W_in'], p['b_in'])

    orders = _bfs_order_sc(eidx)

    for lp in p['layers']:
        aggr = _gine_msg_sc(h, e, src, dst)
        c_eps = (1.0 + lp['eps'])[None]
        h = _gine_layer_tc(h, aggr, c_eps, lp['W1'], lp['b1'],
                           lp['W2'], lp['b2'], lp['gamma'], lp['beta'])

    offs = (jnp.arange(B, dtype=jnp.int32) * T)[:, None]
    perm = orders.reshape(B, T) + offs
    perm_t = perm.T.reshape(-1)
    npad = 32 * 80 * ((n + 32 * 80 - 1) // (32 * 80)) - n
    perm_t_p = jnp.concatenate(
        [perm_t, jnp.zeros((npad,), jnp.int32)])
    h_ord_t = _gather_rows_sc(h, perm_t_p)[:n]

    Y1, Ype, gmean = _gi_tc(h_ord_t, p['pe'][:T], p['W_ih'].T, p['b_ih'])
    out = _gru_tc(Y1.reshape(T, B, 3 * HD), Ype, gmean,
                  p['W_hh'].T, p['b_hh'], p['W_out'], p['b_out'])
    return out


# speedup vs baseline: 1.0589x; 1.0589x over previous
import functools

import jax
import jax.numpy as jnp
from jax import lax
from jax.experimental import pallas as pl
from jax.experimental.pallas import tpu as pltpu
from jax.experimental.pallas import tpu_sc as plsc

NODES_PER = 200
N_GRAPHS_C = 50
EDGES_PER_G = 3200
NP_PAD = 208
HD = 128


def _bfs_order_sc(edge_index):
    n = NODES_PER
    e = EDGES_PER_G
    ng = N_GRAPHS_C
    mesh = plsc.VectorSubcoreMesh(core_axis_name="c", subcore_axis_name="s")
    nw = 32
    rounds = (ng + nw - 1) // nw

    @functools.partial(
        pl.kernel,
        out_type=jax.ShapeDtypeStruct((ng * n,), jnp.int32),
        mesh=mesh,
        compiler_params=pltpu.CompilerParams(needs_layout_passes=False),
        scratch_types=[
            pltpu.VMEM((e,), jnp.int32),
            pltpu.VMEM((e,), jnp.int32),
            pltpu.VMEM((e + 16,), jnp.int32),
            pltpu.VMEM((NP_PAD + 32,), jnp.int32),
            pltpu.SMEM((n + 1,), jnp.int32),
            pltpu.SMEM((n,), jnp.int32),
            pltpu.SMEM((n,), jnp.int32),
        ],
    )
    def k(eidx, out, src_v, dst_v, csr_v, ord_v, ptr_s, deg_s, vis_s):
        cid = lax.axis_index("c")
        sid = lax.axis_index("s")
        wid = sid * 2 + cid

        lane0 = jnp.arange(16, dtype=jnp.int32) == 0

        def scat1(ref, pos, val):
            pvec = jnp.broadcast_to(pos, (16,)).astype(jnp.int32)
            vvec = jnp.broadcast_to(val, (16,)).astype(jnp.int32)
            plsc.store_scatter(ref, [pvec], vvec, mask=lane0)

        def do_graph(g):
            base = g * n
            pltpu.sync_copy(eidx.at[0, pl.ds(g * e, e)], src_v)
            pltpu.sync_copy(eidx.at[1, pl.ds(g * e, e)], dst_v)

            def zblk(v, _):
                deg_s[v] = jnp.int32(0)
                vis_s[v] = jnp.int32(0)
                return 0
            lax.fori_loop(0, n, zblk, 0)

            def hist(i, _):
                s16 = src_v[pl.ds(i * 16, 16)] - base
                for l in range(16):
                    b = s16[l]
                    deg_s[b] = deg_s[b] + 1
                return 0
            lax.fori_loop(0, e // 16, hist, 0)

            def amax(v, carry):
                best, s0 = carry
                d = deg_s[v]
                better = d > best
                return (jnp.where(better, d, best),
                        jnp.where(better, v, s0))
            _, s0 = lax.fori_loop(0, n, amax,
                                  (jnp.int32(-1), jnp.int32(0)))

            ptr_s[0] = jnp.int32(0)

            def pfx(v, c):
                d = deg_s[v]
                c2 = c + d
                ptr_s[v + 1] = c2
                deg_s[v] = c2 - d
                return c2
            lax.fori_loop(0, n, pfx, jnp.int32(0))

            def place(i, _):
                s16 = src_v[pl.ds(i * 16, 16)] - base
                d16 = dst_v[pl.ds(i * 16, 16)] - base
                for l in range(16):
                    b = s16[l]
                    pos = deg_s[b]
                    scat1(csr_v, pos, d16[l])
                    deg_s[b] = pos + 1
                return 0
            lax.fori_loop(0, e // 16, place, 0)

            vis_s[s0] = jnp.int32(1)
            scat1(ord_v, 0, s0)

            def visit(i, tail):
                u_raw = ord_v[pl.ds(i, 16)][0]
                active = i < tail
                u = jnp.where(active, u_raw, 0)
                lo = jnp.where(active, ptr_s[u], 0)
                hi = jnp.where(active, ptr_s[u + 1], 0)

                def scan_edge(carry):
                    ei, t = carry
                    v = csr_v[pl.ds(ei, 16)][0]
                    was = vis_s[v]
                    vis_s[v] = jnp.int32(1)
                    slot = jnp.where(was == 1, jnp.int32(n), t)
                    scat1(ord_v, slot, v)
                    return (ei + 1, t + (1 - was))

                _, tail2 = lax.while_loop(lambda c: c[0] < hi, scan_edge,
                                          (lo, tail))
                return tail2

            tail = lax.fori_loop(0, n, visit, jnp.int32(1))

            def fill(v, t):
                was = vis_s[v]
                slot = jnp.where(was == 1, jnp.int32(n), t)
                scat1(ord_v, slot, v)
                return t + (1 - was)
            lax.fori_loop(0, n, fill, tail)

            pltpu.sync_copy(ord_v.at[pl.ds(0, n)], out.at[pl.ds(g * n, n)])

        for r in range(rounds):
            g = wid + r * nw

            @pl.when(g < ng)
            def _():
                do_graph(g)

    return k(edge_index)


def _gine_msg_sc(h, e_feat, src_arr, dst_arr):
    n = NODES_PER
    epg = EDGES_PER_G
    ng = N_GRAPHS_C
    mesh = plsc.VectorSubcoreMesh(core_axis_name="c", subcore_axis_name="s")
    nw = 32
    rounds = (ng + nw - 1) // nw
    ce = 128
    nch = epg // ce

    @functools.partial(
        pl.kernel,
        out_type=jax.ShapeDtypeStruct((ng * n, HD), jnp.float32),
        mesh=mesh,
        compiler_params=pltpu.CompilerParams(needs_layout_passes=False),
        scratch_types=[
            pltpu.VMEM((ce, HD), jnp.float32),
            pltpu.VMEM((ce, HD), jnp.float32),
            pltpu.VMEM((ce, HD), jnp.float32),
            pltpu.VMEM((ce, HD), jnp.float32),
            pltpu.VMEM((ce, HD), jnp.float32),
            pltpu.VMEM((ce, HD), jnp.float32),
            pltpu.VMEM((40, HD), jnp.float32),
            pltpu.VMEM((ce,), jnp.int32),
            pltpu.VMEM((ce,), jnp.int32),
            pltpu.VMEM((ce,), jnp.int32),
            pltpu.VMEM((ce,), jnp.int32),
            pltpu.VMEM((ce,), jnp.int32),
            pltpu.VMEM((ce,), jnp.int32),
            pltpu.VMEM_SHARED((16 * n, HD), jnp.float32),
            pltpu.SemaphoreType.DMA,
            pltpu.SemaphoreType.DMA,
            pltpu.SemaphoreType.DMA,
            pltpu.SemaphoreType.DMA,
            pltpu.SemaphoreType.DMA,
            pltpu.SemaphoreType.DMA,
        ],
    )
    def k(h_hbm, e_hbm, src_h, dst_h, out, hg0, hg1, e0, e1, msg0, msg1, zero_v,
          sidx0, sidx1, draw0, draw1, didx0, didx1, ag_sh,
          semi0, semi1, semg0, semg1, sems0, sems1):
        cid = lax.axis_index("c")
        sid = lax.axis_index("s")
        wid = sid * 2 + cid
        hg_v = (hg0, hg1)
        e_v = (e0, e1)
        msg_v = (msg0, msg1)
        sidx_v = (sidx0, sidx1)
        draw_v = (draw0, draw1)
        didx_v = (didx0, didx1)
        semi = (semi0, semi1)
        semg = (semg0, semg1)
        sems = (sems0, sems1)

        zeros16 = jnp.zeros((16,), jnp.float32)

        @plsc.parallel_loop(0, 40)
        def zrow(r):
            for c in range(HD // 16):
                zero_v[r, pl.ds(c * 16, 16)] = zeros16

        def issue_in(g, cidx, b):
            eoff = g * epg + cidx * ce
            pltpu.async_copy(src_h.at[pl.ds(eoff, ce)], sidx_v[b],
                             semi[b])
            pltpu.async_copy(dst_h.at[pl.ds(eoff, ce)], draw_v[b],
                             semi[b])
            pltpu.async_copy(e_hbm.at[pl.ds(eoff, ce)], e_v[b], semi[b])

        def wait_in(g, cidx, b):
            eoff = g * epg + cidx * ce
            pltpu.make_async_copy(src_h.at[pl.ds(eoff, ce)],
                                  sidx_v[b], semi[b]).wait()
            pltpu.make_async_copy(dst_h.at[pl.ds(eoff, ce)],
                                  draw_v[b], semi[b]).wait()
            pltpu.make_async_copy(e_hbm.at[pl.ds(eoff, ce)], e_v[b],
                                  semi[b]).wait()

        def wait_scat(b):
            pltpu.make_async_copy(msg_v[b], ag_sh.at[pl.ds(0, ce)],
                                  sems[b]).wait()

        def issue_gather(b):
            pltpu.async_copy(h_hbm.at[sidx_v[b]], hg_v[b], semg[b])

        def wait_gather(b):
            pltpu.make_async_copy(h_hbm.at[sidx_v[b]], hg_v[b],
                                  semg[b]).wait()

        def do_graph(g):
            base = g * n
            arow = sid * n
            for z in range(n // 40):
                pltpu.sync_copy(zero_v, ag_sh.at[pl.ds(arow + z * 40, 40)])
            shift = arow - base

            issue_in(g, 0, 0)
            wait_in(g, 0, 0)
            issue_gather(0)
            issue_in(g, 1, 1)

            def chunk(cidx, _):
                for b in range(2):

                    @pl.when((cidx & 1) == b)
                    def _():
                        @pl.when(cidx + 1 < nch)
                        def _():
                            wait_in(g, cidx + 1, 1 - b)
                            issue_gather(1 - b)

                        @plsc.parallel_loop(0, ce // 16)
                        def dloc(i):
                            sl = pl.ds(i * 16, 16)
                            didx_v[b][sl] = draw_v[b][sl] + shift

                        @pl.when(cidx >= 2)
                        def _():
                            wait_scat(b)

                        wait_gather(b)

                        @plsc.parallel_loop(0, ce)
                        def mrow(r):
                            for c in range(HD // 16):
                                sl = pl.ds(c * 16, 16)
                                msg_v[b][r, sl] = jnp.maximum(
                                    hg_v[b][r, sl] + e_v[b][r, sl], 0.0)

                        pltpu.async_copy(msg_v[b],
                                         ag_sh.at[didx_v[b]],
                                         sems[b], add=True)

                        @pl.when(cidx + 2 < nch)
                        def _():
                            issue_in(g, cidx + 2, b)
                return 0
            lax.fori_loop(0, nch, chunk, 0)

            wait_scat(nch & 1)
            wait_scat(1 - (nch & 1))
            pltpu.sync_copy(ag_sh.at[pl.ds(arow, n)], out.at[pl.ds(base, n)])

        for r in range(rounds):
            g = wid + r * nw

            @pl.when(g < ng)
            def _():
                do_graph(g)

    return k(h, e_feat, src_arr, dst_arr)


def _gather_rows_sc(table, idx):
    nrow, d = idx.shape[0], table.shape[1]
    mesh = plsc.VectorSubcoreMesh(core_axis_name="c", subcore_axis_name="s")
    nw = 32
    rows_per = nrow // nw
    chunk = 80
    nch = rows_per // chunk

    @functools.partial(
        pl.kernel,
        out_type=jax.ShapeDtypeStruct((nrow, d), jnp.float32),
        mesh=mesh,
        compiler_params=pltpu.CompilerParams(needs_layout_passes=False),
        scratch_types=[
            pltpu.VMEM((chunk,), jnp.int32),
            pltpu.VMEM((chunk, d), jnp.float32),
            pltpu.SemaphoreType.DMA,
        ],
    )
    def k(tab, iv, out, idx_v, rows_v, sem):
        cid = lax.axis_index("c")
        sid = lax.axis_index("s")
        wid = sid * 2 + cid
        base = wid * rows_per

        def body(c, _):
            off = base + c * chunk
            pltpu.sync_copy(iv.at[pl.ds(off, chunk)], idx_v)
            pltpu.async_copy(tab.at[idx_v], rows_v, sem).wait()
            pltpu.sync_copy(rows_v, out.at[pl.ds(off, chunk)])
            return 0
        lax.fori_loop(0, nch, body, 0)

    return k(table, idx)


def _edge_mlp_tc(ea, We1, be1, We2, be2):
    E = ea.shape[0]
    blk = 2000
    grid = E // blk

    def body(ea_ref, w1_ref, b1_ref, w2_ref, b2_ref, out_ref):
        a = jnp.maximum(
            jnp.dot(ea_ref[...], w1_ref[...],
                    preferred_element_type=jnp.float32) + b1_ref[...], 0.0)
        out_ref[...] = (
            jnp.dot(a, w2_ref[...], preferred_element_type=jnp.float32)
            + b2_ref[...])

    return pl.pallas_call(
        body,
        grid=(grid,),
        in_specs=[
            pl.BlockSpec((blk, 16), lambda i: (i, 0)),
            pl.BlockSpec((16, HD), lambda i: (0, 0)),
            pl.BlockSpec((1, HD), lambda i: (0, 0)),
            pl.BlockSpec((HD, HD), lambda i: (0, 0)),
            pl.BlockSpec((1, HD), lambda i: (0, 0)),
        ],
        out_specs=pl.BlockSpec((blk, HD), lambda i: (i, 0)),
        out_shape=jax.ShapeDtypeStruct((E, HD), jnp.float32),
    )(ea, We1, be1[None, :], We2, be2[None, :])


def _node_in_tc(x, W, b):
    N = x.shape[0]

    def body(x_ref, w_ref, b_ref, out_ref):
        out_ref[...] = jnp.dot(
            x_ref[...], w_ref[...],
            preferred_element_type=jnp.float32) + b_ref[...]

    return pl.pallas_call(
        body,
        grid=(5,),
        in_specs=[
            pl.BlockSpec((N // 5, HD), lambda i: (i, 0)),
            pl.BlockSpec((HD, HD), lambda i: (0, 0)),
            pl.BlockSpec((1, HD), lambda i: (0, 0)),
        ],
        out_specs=pl.BlockSpec((N // 5, HD), lambda i: (i, 0)),
        out_shape=jax.ShapeDtypeStruct((N, HD), jnp.float32),
    )(x, W, b[None, :])


def _gine_layer_tc(h, aggr, c_eps, W1, b1, W2, b2, gamma, beta):
    N = h.shape[0]

    def body(h_ref, a_ref, ce_ref, w1_ref, b1_ref, w2_ref, b2_ref,
             g_ref, be_ref, out_ref):
        ce = ce_ref[0]
        out = ce * h_ref[...] + a_ref[...]
        a1 = jnp.maximum(
            jnp.dot(out, w1_ref[...],
                    preferred_element_type=jnp.float32) + b1_ref[...], 0.0)
        h2 = jnp.dot(a1, w2_ref[...],
                     preferred_element_type=jnp.float32) + b2_ref[...]
        mean = jnp.mean(h2, axis=0, keepdims=True)
        var = jnp.mean(jnp.square(h2), axis=0, keepdims=True) - mean * mean
        h2n = (g_ref[...] * (h2 - mean) * jax.lax.rsqrt(var + 1e-5)
               + be_ref[...])
        out_ref[...] = jnp.maximum(h2n, 0.0) + h_ref[...]

    return pl.pallas_call(
        body,
        in_specs=[
            pl.BlockSpec(memory_space=pltpu.VMEM),
            pl.BlockSpec(memory_space=pltpu.VMEM),
            pl.BlockSpec(memory_space=pltpu.SMEM),
            pl.BlockSpec(memory_space=pltpu.VMEM),
            pl.BlockSpec(memory_space=pltpu.VMEM),
            pl.BlockSpec(memory_space=pltpu.VMEM),
            pl.BlockSpec(memory_space=pltpu.VMEM),
            pl.BlockSpec(memory_space=pltpu.VMEM),
            pl.BlockSpec(memory_space=pltpu.VMEM),
        ],
        out_shape=jax.ShapeDtypeStruct((N, HD), jnp.float32),
    )(h, aggr, c_eps, W1, b1[None, :], W2, b2[None, :],
      gamma[None, :], beta[None, :])


def _gi_tc(h_ord_t, pe, W_ihT, b_ih):
    TB = h_ord_t.shape[0]
    T = pe.shape[0]
    B = TB // T

    def body(x_ref, pe_ref, w_ref, b_ref, y1_ref, ype_ref, gm_ref):
        x = x_ref[...]
        y1_ref[...] = jnp.dot(
            x, w_ref[...], preferred_element_type=jnp.float32) + b_ref[...]
        ype_ref[...] = jnp.dot(
            pe_ref[...], w_ref[...], preferred_element_type=jnp.float32)
        gm_ref[...] = jnp.mean(x.reshape(T, B, HD), axis=0)

    return pl.pallas_call(
        body,
        out_shape=(
            jax.ShapeDtypeStruct((TB, 3 * HD), jnp.float32),
            jax.ShapeDtypeStruct((T, 3 * HD), jnp.float32),
            jax.ShapeDtypeStruct((B, HD), jnp.float32),
        ),
    )(h_ord_t, pe, W_ihT, b_ih[None, :])


def _gru_tc(Y1, Ype, gmean, W_hhT, b_hh, W_out, b_out):
    T, B = Y1.shape[0], Y1.shape[1]

    def body(y1_ref, ype_ref, gm_ref, whh_ref, bhh_ref, wo_ref, bo_ref,
             out_ref):
        whh = whh_ref[...]
        bhh = bhh_ref[...]

        def step(t, carry):
            h, s = carry
            gi = (y1_ref[pl.ds(t, 1)].reshape(B, 3 * HD)
                  + ype_ref[pl.ds(t, 1)])
            gh = jnp.dot(h, whh, preferred_element_type=jnp.float32) + bhh
            ir, iz, inn = gi[:, :HD], gi[:, HD:2 * HD], gi[:, 2 * HD:]
            hr, hz, hn = gh[:, :HD], gh[:, HD:2 * HD], gh[:, 2 * HD:]
            r = jax.nn.sigmoid(ir + hr)
            z = jax.nn.sigmoid(iz + hz)
            nn_ = jnp.tanh(inn + r * hn)
            hnew = (1.0 - z) * nn_ + z * h
            return (hnew, s + hnew)

        h0 = jnp.zeros((B, HD), jnp.float32)
        h_last, s = lax.fori_loop(0, T, step, (h0, h0))
        pooled = jnp.concatenate(
            [h_last, s * (1.0 / T), gm_ref[...]], axis=1)
        out_ref[...] = jnp.dot(
            pooled, wo_ref[...], preferred_element_type=jnp.float32
        ) + bo_ref[...]

    return pl.pallas_call(
        body,
        out_shape=jax.ShapeDtypeStruct((B, HD), jnp.float32),
    )(Y1, Ype, gmean, W_hhT, b_hh[None, :], W_out, b_out[None, :])


def kernel(x, edge_index, edge_attr, batch_vec, params):
    p = params
    del batch_vec
    n = x.shape[0]
    B = N_GRAPHS_C
    T = NODES_PER
    eidx = edge_index.astype(jnp.int32)
    src = eidx[0]
    dst = eidx[1]

    e = _edge_mlp_tc(edge_attr, p['We1'], p['be1'], p['We2'], p['be2'])
    h = _node_in_tc(x, p['W_in'], p['b_in'])

    orders = _bfs_order_sc(eidx)

    for lp in p['layers']:
        aggr = _gine_msg_sc(h, e, src, dst)
        c_eps = (1.0 + lp['eps'])[None]
        h = _gine_layer_tc(h, aggr, c_eps, lp['W1'], lp['b1'],
                           lp['W2'], lp['b2'], lp['gamma'], lp['beta'])

    offs = (jnp.arange(B, dtype=jnp.int32) * T)[:, None]
    perm = orders.reshape(B, T) + offs
    perm_t = perm.T.reshape(-1)
    npad = 32 * 80 * ((n + 32 * 80 - 1) // (32 * 80)) - n
    perm_t_p = jnp.concatenate(
        [perm_t, jnp.zeros((npad,), jnp.int32)])
    h_ord_t = _gather_rows_sc(h, perm_t_p)[:n]

    Y1, Ype, gmean = _gi_tc(h_ord_t, p['pe'][:T], p['W_ih'].T, p['b_ih'])
    out = _gru_tc(Y1.reshape(T, B, 3 * HD), Ype, gmean,
                  p['W_hh'].T, p['b_hh'], p['W_out'], p['b_out'])
    return out
